# baseline (speedup 1.0000x reference)
import functools
import math

import jax
import jax.numpy as jnp
from jax import lax
from jax.experimental import pallas as pl
from jax.experimental.pallas import tpu as pltpu
from jax.experimental.pallas import tpu_sc as plsc

B = 16384
L = 200
HID = 32
SCALE = math.sqrt(32.0)

IDX_MINOR = 100
CB = 8
ROWS_PER_CHUNK = CB * L
GATHERS_PER_CHUNK = ROWS_PER_CHUNK // IDX_MINOR


def _sc_embed(idx2d, table):
    info = plsc.get_sparse_core_info()
    num_workers = info.num_cores * info.num_subcores
    b_per_w = B // num_workers
    chunks = b_per_w // CB
    idx_rows_per_chunk = ROWS_PER_CHUNK // IDX_MINOR

    mesh = plsc.VectorSubcoreMesh(core_axis_name="c", subcore_axis_name="s")

    @functools.partial(
        pl.kernel,
        mesh=mesh,
        out_type=jax.ShapeDtypeStruct((B, L, HID), jnp.float32),
        compiler_params=pltpu.CompilerParams(use_tc_tiling_on_sc=False),
        scratch_types=[
            pltpu.VMEM((GATHERS_PER_CHUNK, IDX_MINOR), jnp.int32),
            pltpu.VMEM((GATHERS_PER_CHUNK, IDX_MINOR), jnp.int32),
            pltpu.VMEM((ROWS_PER_CHUNK, HID), jnp.float32),
            pltpu.VMEM((ROWS_PER_CHUNK, HID), jnp.float32),
            pltpu.SemaphoreType.DMA,
            pltpu.SemaphoreType.DMA,
            pltpu.SemaphoreType.DMA,
            pltpu.SemaphoreType.DMA,
        ],
    )
    def k(idx_hbm, table_hbm, out_hbm,
          idx_a, idx_b, rows_a, rows_b, gsem_a, gsem_b, ssem_a, ssem_b):
        wid = lax.axis_index("s") * info.num_cores + lax.axis_index("c")
        b0 = wid * b_per_w
        r0 = b0 * (L // IDX_MINOR)

        idx_bufs = (idx_a, idx_b)
        row_bufs = (rows_a, rows_b)
        gsems = (gsem_a, gsem_b)
        ssems = (ssem_a, ssem_b)

        def fire_chunk(c, slot):
            pltpu.sync_copy(
                idx_hbm.at[pl.ds(r0 + c * idx_rows_per_chunk,
                                 idx_rows_per_chunk)],
                idx_bufs[slot])
            for t in range(GATHERS_PER_CHUNK):
                pltpu.make_async_copy(
                    table_hbm.at[idx_bufs[slot].at[t]],
                    row_bufs[slot].at[pl.ds(t * IDX_MINOR, IDX_MINOR)],
                    gsems[slot]).start()

        def wait_gathers(slot):
            for t in range(GATHERS_PER_CHUNK):
                pltpu.make_async_copy(
                    table_hbm.at[idx_bufs[slot].at[t]],
                    row_bufs[slot].at[pl.ds(t * IDX_MINOR, IDX_MINOR)],
                    gsems[slot]).wait()


        def fire_out(c, slot):
            for bi in range(CB):
                pltpu.make_async_copy(
                    row_bufs[slot].at[pl.ds(bi * L, L)],
                    out_hbm.at[b0 + c * CB + bi],
                    ssems[slot]).start()

        def wait_out(c, slot):
            for bi in range(CB):
                pltpu.make_async_copy(
                    row_bufs[slot].at[pl.ds(bi * L, L)],
                    out_hbm.at[b0 + c * CB + bi],
                    ssems[slot]).wait()

        fire_chunk(0, 0)

        def body(s, carry):
            c0 = 2 * s
            c1 = c0 + 1

            @pl.when(s > 0)
            def _():
                wait_out(c1 - 2, 1)
            fire_chunk(c1, 1)

            wait_gathers(0)
            fire_out(c0, 0)

            @pl.when(s < chunks // 2 - 1)
            def _():
                wait_out(c0, 0)
                fire_chunk(c0 + 2, 0)

            wait_gathers(1)
            fire_out(c1, 1)
            return carry

        lax.fori_loop(0, chunks // 2, body, 0)
        wait_out(chunks - 2, 0)
        wait_out(chunks - 1, 1)

    return k(idx2d, table)


BB = 128


def _tc_transpose_scale(raw):
    v = raw.reshape(B * L // 4 // 128 * 128 // 128 * 128 // 128, 128)         if False else raw.reshape(B * (L // 4), 128)

    def body(r_ref, o_ref):
        x3 = r_ref[...].reshape(BB, L // 4, 128)
        for lg0 in (0, 8, 16, 24, 32, 40):
            z = jnp.transpose(x3[:, lg0:lg0 + 8, :], (1, 2, 0))
            o_ref[pl.ds(lg0 * 4, 32)] = (z * SCALE).reshape(32, HID, BB)
        z = jnp.transpose(x3[:, 48:50, :], (1, 2, 0))
        o_ref[pl.ds(192, 8)] = (z * SCALE).reshape(8, HID, BB)

    return pl.pallas_call(
        body,
        grid=(B // BB,),
        in_specs=[pl.BlockSpec((BB * (L // 4), 128), lambda i: (i, 0))],
        out_specs=pl.BlockSpec((L, HID, BB), lambda i: (0, 0, i)),
        out_shape=jax.ShapeDtypeStruct((L, HID, B), jnp.float32),
    )(v)


def kernel(x, table):
    idx2d = x.reshape(B * L // IDX_MINOR, IDX_MINOR).astype(jnp.int32)
    raw = _sc_embed(idx2d, table)
    out_t = _tc_transpose_scale(raw)
    return jnp.transpose(out_t, (2, 0, 1))


# speedup vs baseline: 4.0286x; 4.0286x over previous
import functools
import math

import jax
import jax.numpy as jnp
from jax import lax
from jax.experimental import pallas as pl
from jax.experimental.pallas import tpu as pltpu
from jax.experimental.pallas import tpu_sc as plsc

B = 16384
L = 200
HID = 32
SCALE = math.sqrt(32.0)

IDX_MINOR = 100
CB = 8
ROWS_PER_CHUNK = CB * L
GATHERS_PER_CHUNK = ROWS_PER_CHUNK // IDX_MINOR


def _sc_embed(idx2d, table):
    info = plsc.get_sparse_core_info()
    num_workers = info.num_cores * info.num_subcores
    b_per_w = B // num_workers
    chunks = b_per_w // CB
    idx_rows_per_chunk = ROWS_PER_CHUNK // IDX_MINOR

    mesh = plsc.VectorSubcoreMesh(core_axis_name="c", subcore_axis_name="s")

    @functools.partial(
        pl.kernel,
        mesh=mesh,
        out_type=jax.ShapeDtypeStruct((B, L, HID), jnp.float32),
        compiler_params=pltpu.CompilerParams(use_tc_tiling_on_sc=False),
        scratch_types=[
            pltpu.VMEM((GATHERS_PER_CHUNK, IDX_MINOR), jnp.int32),
            pltpu.VMEM((GATHERS_PER_CHUNK, IDX_MINOR), jnp.int32),
            pltpu.VMEM((ROWS_PER_CHUNK, HID), jnp.float32),
            pltpu.VMEM((ROWS_PER_CHUNK, HID), jnp.float32),
            pltpu.SemaphoreType.DMA,
            pltpu.SemaphoreType.DMA,
            pltpu.SemaphoreType.DMA,
            pltpu.SemaphoreType.DMA,
        ],
    )
    def k(idx_hbm, table_hbm, out_hbm,
          idx_a, idx_b, rows_a, rows_b, gsem_a, gsem_b, ssem_a, ssem_b):
        wid = lax.axis_index("s") * info.num_cores + lax.axis_index("c")
        b0 = wid * b_per_w
        r0 = b0 * (L // IDX_MINOR)

        idx_bufs = (idx_a, idx_b)
        row_bufs = (rows_a, rows_b)
        gsems = (gsem_a, gsem_b)
        ssems = (ssem_a, ssem_b)

        def fire_chunk(c, slot):
            pltpu.sync_copy(
                idx_hbm.at[pl.ds(r0 + c * idx_rows_per_chunk,
                                 idx_rows_per_chunk)],
                idx_bufs[slot])
            for t in range(GATHERS_PER_CHUNK):
                pltpu.make_async_copy(
                    table_hbm.at[idx_bufs[slot].at[t]],
                    row_bufs[slot].at[pl.ds(t * IDX_MINOR, IDX_MINOR)],
                    gsems[slot]).start()

        def wait_gathers(slot):
            for t in range(GATHERS_PER_CHUNK):
                pltpu.make_async_copy(
                    table_hbm.at[idx_bufs[slot].at[t]],
                    row_bufs[slot].at[pl.ds(t * IDX_MINOR, IDX_MINOR)],
                    gsems[slot]).wait()


        def fire_out(c, slot):
            for bi in range(CB):
                pltpu.make_async_copy(
                    row_bufs[slot].at[pl.ds(bi * L, L)],
                    out_hbm.at[b0 + c * CB + bi],
                    ssems[slot]).start()

        def wait_out(c, slot):
            for bi in range(CB):
                pltpu.make_async_copy(
                    row_bufs[slot].at[pl.ds(bi * L, L)],
                    out_hbm.at[b0 + c * CB + bi],
                    ssems[slot]).wait()

        fire_chunk(0, 0)

        def body(s, carry):
            c0 = 2 * s
            c1 = c0 + 1

            @pl.when(s > 0)
            def _():
                wait_out(c1 - 2, 1)
            fire_chunk(c1, 1)

            wait_gathers(0)
            fire_out(c0, 0)

            @pl.when(s < chunks // 2 - 1)
            def _():
                wait_out(c0, 0)
                fire_chunk(c0 + 2, 0)

            wait_gathers(1)
            fire_out(c1, 1)
            return carry

        lax.fori_loop(0, chunks // 2, body, 0)
        wait_out(chunks - 2, 0)
        wait_out(chunks - 1, 1)

    return k(idx2d, table)


BB = 128


def _tc_transpose_scale(raw):
    v = raw.reshape(B * L // 4 // 128 * 128 // 128 * 128 // 128, 128)         if False else raw.reshape(B * (L // 4), 128)

    def body(r_ref, o_ref):
        x3 = r_ref[...].reshape(BB, L // 4, 128)
        for lg in range(L // 4):
            t = jnp.transpose(x3[:, lg, :])
            o_ref[pl.ds(lg * 4, 4)] = (t * SCALE).reshape(4, HID, BB)

    return pl.pallas_call(
        body,
        grid=(B // BB,),
        in_specs=[pl.BlockSpec((BB * (L // 4), 128), lambda i: (i, 0))],
        out_specs=pl.BlockSpec((L, HID, BB), lambda i: (0, 0, i)),
        out_shape=jax.ShapeDtypeStruct((L, HID, B), jnp.float32),
    )(v)


def kernel(x, table):
    idx2d = x.reshape(B * L // IDX_MINOR, IDX_MINOR).astype(jnp.int32)
    raw = _sc_embed(idx2d, table)
    out_t = _tc_transpose_scale(raw)
    return jnp.transpose(out_t, (2, 0, 1))


# speedup vs baseline: 4.0287x; 1.0000x over previous
import functools
import math

import jax
import jax.numpy as jnp
from jax import lax
from jax.experimental import pallas as pl
from jax.experimental.pallas import tpu as pltpu
from jax.experimental.pallas import tpu_sc as plsc

B = 16384
L = 200
HID = 32
SCALE = math.sqrt(32.0)

IDX_MINOR = 100
CB = 8
ROWS_PER_CHUNK = CB * L
GATHERS_PER_CHUNK = ROWS_PER_CHUNK // IDX_MINOR


def _sc_embed(idx2d, table):
    info = plsc.get_sparse_core_info()
    num_workers = info.num_cores * info.num_subcores
    b_per_w = B // num_workers
    chunks = b_per_w // CB
    idx_rows_per_chunk = ROWS_PER_CHUNK // IDX_MINOR

    mesh = plsc.VectorSubcoreMesh(core_axis_name="c", subcore_axis_name="s")

    @functools.partial(
        pl.kernel,
        mesh=mesh,
        out_type=jax.ShapeDtypeStruct((B, L, HID), jnp.float32),
        compiler_params=pltpu.CompilerParams(use_tc_tiling_on_sc=False),
        scratch_types=[
            pltpu.VMEM((GATHERS_PER_CHUNK, IDX_MINOR), jnp.int32),
            pltpu.VMEM((GATHERS_PER_CHUNK, IDX_MINOR), jnp.int32),
            pltpu.VMEM((ROWS_PER_CHUNK, HID), jnp.float32),
            pltpu.VMEM((ROWS_PER_CHUNK, HID), jnp.float32),
            pltpu.SemaphoreType.DMA,
            pltpu.SemaphoreType.DMA,
            pltpu.SemaphoreType.DMA,
            pltpu.SemaphoreType.DMA,
        ],
    )
    def k(idx_hbm, table_hbm, out_hbm,
          idx_a, idx_b, rows_a, rows_b, gsem_a, gsem_b, ssem_a, ssem_b):
        wid = lax.axis_index("s") * info.num_cores + lax.axis_index("c")
        b0 = wid * b_per_w
        r0 = b0 * (L // IDX_MINOR)

        idx_bufs = (idx_a, idx_b)
        row_bufs = (rows_a, rows_b)
        gsems = (gsem_a, gsem_b)
        ssems = (ssem_a, ssem_b)

        def fire_chunk(c, slot):
            pltpu.sync_copy(
                idx_hbm.at[pl.ds(r0 + c * idx_rows_per_chunk,
                                 idx_rows_per_chunk)],
                idx_bufs[slot])
            for t in range(GATHERS_PER_CHUNK):
                pltpu.make_async_copy(
                    table_hbm.at[idx_bufs[slot].at[t]],
                    row_bufs[slot].at[pl.ds(t * IDX_MINOR, IDX_MINOR)],
                    gsems[slot]).start()

        def wait_gathers(slot):
            for t in range(GATHERS_PER_CHUNK):
                pltpu.make_async_copy(
                    table_hbm.at[idx_bufs[slot].at[t]],
                    row_bufs[slot].at[pl.ds(t * IDX_MINOR, IDX_MINOR)],
                    gsems[slot]).wait()


        def fire_out(c, slot):
            for bi in range(CB):
                pltpu.make_async_copy(
                    row_bufs[slot].at[pl.ds(bi * L, L)],
                    out_hbm.at[b0 + c * CB + bi],
                    ssems[slot]).start()

        def wait_out(c, slot):
            for bi in range(CB):
                pltpu.make_async_copy(
                    row_bufs[slot].at[pl.ds(bi * L, L)],
                    out_hbm.at[b0 + c * CB + bi],
                    ssems[slot]).wait()

        fire_chunk(0, 0)

        def body(s, carry):
            c0 = 2 * s
            c1 = c0 + 1

            @pl.when(s > 0)
            def _():
                wait_out(c1 - 2, 1)
            fire_chunk(c1, 1)

            wait_gathers(0)
            fire_out(c0, 0)

            @pl.when(s < chunks // 2 - 1)
            def _():
                wait_out(c0, 0)
                fire_chunk(c0 + 2, 0)

            wait_gathers(1)
            fire_out(c1, 1)
            return carry

        lax.fori_loop(0, chunks // 2, body, 0)
        wait_out(chunks - 2, 0)
        wait_out(chunks - 1, 1)

    return k(idx2d, table)


BB = 128


def _tc_transpose_scale(raw):
    v = raw.reshape(B * (L // 4), 128)

    def body(r_ref, o_ref):
        x3 = r_ref[...].reshape(BB, L // 4, 128)
        for lg in range(L // 4):
            t = jnp.transpose(x3[:, lg, :])
            o_ref[pl.ds(lg * 4, 4)] = (t * SCALE).reshape(4, HID, BB)

    return pl.pallas_call(
        body,
        grid=(B // BB,),
        in_specs=[pl.BlockSpec((BB * (L // 4), 128), lambda i: (i, 0))],
        out_specs=pl.BlockSpec((L, HID, BB), lambda i: (0, 0, i)),
        out_shape=jax.ShapeDtypeStruct((L, HID, B), jnp.float32),
    )(v)


def kernel(x, table):
    idx2d = x.reshape(B * L // IDX_MINOR, IDX_MINOR).astype(jnp.int32)
    raw = _sc_embed(idx2d, table)
    out_t = _tc_transpose_scale(raw)
    return jnp.transpose(out_t, (2, 0, 1))
